# baseline (speedup 1.0000x reference)
import functools

import jax
import jax.numpy as jnp
from jax import lax
from jax.experimental import pallas as pl
from jax.experimental.pallas import tpu as pltpu
import jax.experimental.pallas.tpu_sc as plsc

N = 10000
E = 320000
D = 128
DO = 128
DREAL = 50
B = 1024
WD = 5e-4

NC, NS = 2, 16
NW = NC * NS
RT = 640
NPAD = RT * NS
DUMMY = N + 8
EW = E // NW
CH = 128
KR = 80
HS = KR // 2
FR = 16
BT = B // NW

_mesh = plsc.VectorSubcoreMesh(core_axis_name="c", subcore_axis_name="s",
                               num_cores=NC, num_subcores=NS)


def _sc_msg_body(feats, srcp, dstp, z1d, z2d,
                 feats2, aggp, rs_out,
                 src_c, dst_c, rows_a, rows_b, degv, rsv,
                 agg_s, deg_s, sema, semb, semc, semd):
    cid = lax.axis_index("c")
    sid = lax.axis_index("s")
    wid = sid * NC + cid
    r0 = sid * RT

    pltpu.sync_copy(z2d, rows_a)
    pltpu.sync_copy(z1d.at[pl.ds(r0, RT)], degv)
    pltpu.sync_copy(degv, deg_s.at[pl.ds(r0, RT)])
    for j in range(RT // CH):
        pltpu.sync_copy(rows_a, agg_s.at[pl.ds(r0 + j * CH, CH)])
    ones16 = jnp.full((16,), 1.0, jnp.float32)
    for i in range(CH // 16):
        rows_b[0, pl.ds(i * 16, 16)] = ones16
    ones_r = rows_b.at[0]
    plsc.subcore_barrier()

    _scope_deg = jax.named_scope("ph_deg")
    _scope_deg.__enter__()
    for w in (0, NS):
        for h in (0, HS):
            pltpu.sync_copy(dstp.at[sid + w, pl.ds(h, HS)], src_c)

            @pl.loop(0, HS)
            def _fire(k):
                pltpu.async_copy(ones_r, deg_s.at[src_c.at[k]], sema,
                                 add=True)

            @pl.loop(0, HS)
            def _drain(k):
                pltpu.make_async_copy(ones_r, deg_s.at[src_c.at[0]],
                                      sema).wait()
    plsc.subcore_barrier()
    _scope_deg.__exit__(None, None, None)

    pltpu.sync_copy(deg_s.at[pl.ds(r0, RT)], degv)

    @pl.loop(0, RT // 16)
    def _rs(i):
        x = jnp.maximum(degv[pl.ds(i * 16, 16)], 1.0)
        h = jnp.int32(0x5F3759DF) - (lax.bitcast_convert_type(x, jnp.int32) >> 1)
        y = lax.bitcast_convert_type(h, jnp.float32)
        y = y * (1.5 - 0.5 * x * y * y)
        y = y * (1.5 - 0.5 * x * y * y)
        y = y * (1.5 - 0.5 * x * y * y)
        rsv[pl.ds(i * 16, 16)] = y

    pltpu.sync_copy(rsv, rs_out.at[cid, pl.ds(r0, RT)])

    _scope_f2 = jax.named_scope("ph_feats2")
    _scope_f2.__enter__()
    nrows = jnp.minimum(RT, N - r0)

    @pl.loop(0, nrows // FR)
    def _scale(k):
        g0 = r0 + k * FR
        pltpu.sync_copy(feats.at[pl.ds(g0, FR)], rows_b.at[pl.ds(0, FR)])
        rv = rsv[pl.ds(k * FR, 16)]
        for r in range(16):
            sv = lax.broadcast(rv[r], (16,))
            for v in range(D // 16):
                rows_b[r, pl.ds(v * 16, 16)] = (
                    rows_b[r, pl.ds(v * 16, 16)] * sv)
        pltpu.sync_copy(rows_b.at[pl.ds(0, FR)], feats2.at[pl.ds(g0, FR)])
    plsc.subcore_barrier()
    _scope_f2.__exit__(None, None, None)

    _scope_ed = jax.named_scope("ph_edges")
    _scope_ed.__enter__()

    for h in (0, HS):
        pltpu.sync_copy(srcp.at[wid, pl.ds(h, HS)], src_c)
        pltpu.sync_copy(dstp.at[wid, pl.ds(h, HS)], dst_c)

        @pl.loop(0, HS)
        def _edge(k):
            pltpu.async_copy(feats2.at[src_c.at[k]], rows_a, sema).wait()
            pltpu.sync_copy(rows_a, agg_s.at[dst_c.at[k]], add=True)
    plsc.subcore_barrier()
    _scope_ed.__exit__(None, None, None)

    descs = []
    for j in range(RT // CH):
        buf, sem = (rows_a, sema) if j % 2 == 0 else (rows_b, semb)
        if j >= 2:
            descs[j - 2].wait()
        pltpu.sync_copy(agg_s.at[pl.ds(r0 + j * CH, CH)], buf)
        descs.append(
            pltpu.async_copy(buf, aggp.at[cid, pl.ds(r0 + j * CH, CH)], sem))
    for dsc in descs[-2:]:
        dsc.wait()


_sc_msg = functools.partial(
    pl.kernel,
    out_type=(
        jax.ShapeDtypeStruct((N, D), jnp.float32),
        jax.ShapeDtypeStruct((NC, NPAD, D), jnp.float32),
        jax.ShapeDtypeStruct((NC, NPAD), jnp.float32),
    ),
    mesh=_mesh,
    scratch_types=[
        pltpu.VMEM((HS, CH), jnp.int32),
        pltpu.VMEM((HS, CH), jnp.int32),
        pltpu.VMEM((CH, D), jnp.float32),
        pltpu.VMEM((CH, D), jnp.float32),
        pltpu.VMEM((RT,), jnp.float32),
        pltpu.VMEM((RT,), jnp.float32),
        pltpu.VMEM_SHARED((NPAD, D), jnp.float32),
        pltpu.VMEM_SHARED((NPAD,), jnp.float32),
        pltpu.SemaphoreType.DMA,
        pltpu.SemaphoreType.DMA,
        pltpu.SemaphoreType.DMA,
        pltpu.SemaphoreType.DMA,
    ],
)(_sc_msg_body)


def _tc_embed_body(aggp_ref, rs_ref, w_ref, emb_ref, wd_ref):
    a = aggp_ref[0, :N, :] + aggp_ref[1, :N, :]
    a = a * rs_ref[0, :N][:, None]
    h = jnp.tanh(jnp.dot(a, w_ref[...], preferred_element_type=jnp.float32))
    ss = jnp.sum(h * h, axis=1, keepdims=True)
    e = h * lax.rsqrt(ss + 1e-12)
    emb_ref[...] = e
    wd_ref[...] = jnp.sum(e * e).reshape(1, 1)


_tc_embed = pl.pallas_call(
    _tc_embed_body,
    out_shape=(
        jax.ShapeDtypeStruct((N, DO), jnp.float32),
        jax.ShapeDtypeStruct((1, 1), jnp.float32),
    ),
)


def _sc_gather_body(emb, bidx, outs, idxv, buf, sem):
    cid = lax.axis_index("c")
    sid = lax.axis_index("s")
    wid = sid * NC + cid
    o = wid * BT
    for b in range(3):
        pltpu.sync_copy(bidx.at[b, pl.ds(o, BT)], idxv)
        pltpu.async_copy(emb.at[idxv], buf, sem).wait()
        pltpu.sync_copy(buf, outs.at[b, pl.ds(o, BT)])


_sc_gather = functools.partial(
    pl.kernel,
    out_type=jax.ShapeDtypeStruct((3, B, DO), jnp.float32),
    mesh=_mesh,
    scratch_types=[
        pltpu.VMEM((BT,), jnp.int32),
        pltpu.VMEM((BT, DO), jnp.float32),
        pltpu.SemaphoreType.DMA,
    ],
)(_sc_gather_body)


def _tc_loss_body(o_ref, wd_ref, out_ref):
    o1 = o_ref[0]
    o2 = o_ref[1]
    o3 = o_ref[2]
    y_ui = jnp.sum(o1 * o2, axis=1)
    y_uj = jnp.sum(o1 * o3, axis=1)
    d = y_ui - y_uj
    sig = 1.0 / (1.0 + jnp.exp(-d))
    loss = jnp.sum(-jnp.log(sig + 1e-12))
    loss = loss + WD * 0.5 * wd_ref[0, 0]
    out_ref[...] = (loss / B).reshape(1, 1)


_tc_loss = pl.pallas_call(
    _tc_loss_body,
    out_shape=jax.ShapeDtypeStruct((1, 1), jnp.float32),
)


def kernel(feats, W, edge_index, batch1, batch2, batch3):
    src = edge_index[0].reshape(NW, EW)
    dst = edge_index[1].reshape(NW, EW)
    pad = KR * CH - EW
    srcp = jnp.concatenate(
        [src, jnp.zeros((NW, pad), jnp.int32)], axis=1).reshape(NW, KR, CH)
    dstp = jnp.concatenate(
        [dst, jnp.full((NW, pad), DUMMY, jnp.int32)], axis=1).reshape(NW, KR, CH)
    z1d = jnp.zeros((NPAD,), jnp.float32)
    z2d = jnp.zeros((CH, D), jnp.float32)
    feats2, aggp, rs = _sc_msg(feats, srcp, dstp, z1d, z2d)
    del feats2
    Wp = jnp.zeros((D, DO), jnp.float32).at[:, :DREAL].set(W)
    emb, wd = _tc_embed(aggp, rs, Wp)
    bidx = jnp.stack([batch1, batch2, batch3])
    outs = _sc_gather(emb, bidx)
    loss = _tc_loss(outs, wd)
    return loss[0, 0]


# speedup vs baseline: 1.3895x; 1.3895x over previous
import functools

import jax
import jax.numpy as jnp
from jax import lax
from jax.experimental import pallas as pl
from jax.experimental.pallas import tpu as pltpu
import jax.experimental.pallas.tpu_sc as plsc

N = 10000
E = 320000
D = 128
DO = 128
DREAL = 50
B = 1024
WD = 5e-4

NC, NS = 2, 16
NW = NC * NS
RT = 640
NPAD = RT * NS
DUMMY = N + 8
CH = 128
KW = 79
EWP = KW * CH
BT = B // NW

_mesh = plsc.VectorSubcoreMesh(core_axis_name="c", subcore_axis_name="s",
                               num_cores=NC, num_subcores=NS)


def _sc_msg_body(feats, srcp, dstp, zer, z128, ones,
                 feats2, aggp, rs_out,
                 src_v, dst_v, rows_v, fbuf, degv, rsv, ones_v, zer_v,
                 agg_s, deg_s, sem):
    cid = lax.axis_index("c")
    sid = lax.axis_index("s")
    wid = sid * NC + cid
    r0 = sid * RT

    pltpu.sync_copy(zer, zer_v)
    pltpu.sync_copy(ones, ones_v)
    pltpu.sync_copy(z128, rows_v)
    pltpu.sync_copy(zer_v, deg_s.at[pl.ds(r0, RT)])
    for j in range(RT // CH):
        pltpu.sync_copy(rows_v, agg_s.at[pl.ds(r0 + j * CH, CH)])
    plsc.subcore_barrier()

    for w0 in (0, NS):
        pltpu.sync_copy(dstp.at[sid + w0], dst_v)

        @pl.loop(0, KW)
        def _deg(k):
            pltpu.sync_copy(ones_v, deg_s.at[dst_v.at[k]], add=True)
    plsc.subcore_barrier()

    pltpu.sync_copy(deg_s.at[pl.ds(r0, RT)], degv)

    @pl.loop(0, RT // 16)
    def _rs(i):
        x = jnp.maximum(degv[pl.ds(i * 16, 16)], 1.0)
        h = jnp.int32(0x5F3759DF) - (lax.bitcast_convert_type(x, jnp.int32) >> 1)
        y = lax.bitcast_convert_type(h, jnp.float32)
        y = y * (1.5 - 0.5 * x * y * y)
        y = y * (1.5 - 0.5 * x * y * y)
        y = y * (1.5 - 0.5 * x * y * y)
        rsv[pl.ds(i * 16, 16)] = y

    pltpu.sync_copy(rsv, rs_out.at[cid, pl.ds(r0, RT)])

    nrows = jnp.minimum(RT, N - r0)

    @pl.loop(0, nrows // 16)
    def _scale(k):
        g0 = r0 + k * 16
        pltpu.sync_copy(feats.at[pl.ds(g0, 16)], fbuf)
        rv = rsv[pl.ds(k * 16, 16)]
        for r in range(16):
            sv = lax.broadcast(rv[r], (16,))
            for v in range(D // 16):
                fbuf[r, pl.ds(v * 16, 16)] = fbuf[r, pl.ds(v * 16, 16)] * sv
        pltpu.sync_copy(fbuf, feats2.at[pl.ds(g0, 16)])
    plsc.subcore_barrier()

    pltpu.sync_copy(srcp.at[wid], src_v)
    pltpu.sync_copy(dstp.at[wid], dst_v)

    @pl.loop(0, KW)
    def _edge(k):
        pltpu.async_copy(feats2.at[src_v.at[k]], rows_v, sem).wait()
        pltpu.sync_copy(rows_v, agg_s.at[dst_v.at[k]], add=True)
    plsc.subcore_barrier()

    for j in range(RT // CH):
        pltpu.sync_copy(agg_s.at[pl.ds(r0 + j * CH, CH)], rows_v)
        pltpu.sync_copy(rows_v, aggp.at[cid, pl.ds(r0 + j * CH, CH)])


_sc_msg = functools.partial(
    pl.kernel,
    out_type=(
        jax.ShapeDtypeStruct((N, D), jnp.float32),
        jax.ShapeDtypeStruct((NC, NPAD, D), jnp.float32),
        jax.ShapeDtypeStruct((NC, NPAD), jnp.float32),
    ),
    mesh=_mesh,
    scratch_types=[
        pltpu.VMEM((KW, CH), jnp.int32),
        pltpu.VMEM((KW, CH), jnp.int32),
        pltpu.VMEM((CH, D), jnp.float32),
        pltpu.VMEM((16, D), jnp.float32),
        pltpu.VMEM((RT,), jnp.float32),
        pltpu.VMEM((RT,), jnp.float32),
        pltpu.VMEM((CH,), jnp.float32),
        pltpu.VMEM((RT,), jnp.float32),
        pltpu.VMEM_SHARED((NPAD, D), jnp.float32),
        pltpu.VMEM_SHARED((NPAD,), jnp.float32),
        pltpu.SemaphoreType.DMA,
    ],
)(_sc_msg_body)


def _tc_embed_body(aggp_ref, rs_ref, w_ref, emb_ref, wd_ref):
    a = aggp_ref[0, :N, :] + aggp_ref[1, :N, :]
    a = a * rs_ref[0, :N][:, None]
    h = jnp.tanh(jnp.dot(a, w_ref[...], preferred_element_type=jnp.float32))
    ss = jnp.sum(h * h, axis=1, keepdims=True)
    e = h * lax.rsqrt(ss + 1e-12)
    emb_ref[...] = e
    wd_ref[...] = jnp.sum(e * e).reshape(1, 1)


_tc_embed = pl.pallas_call(
    _tc_embed_body,
    out_shape=(
        jax.ShapeDtypeStruct((N, DO), jnp.float32),
        jax.ShapeDtypeStruct((1, 1), jnp.float32),
    ),
)


def _sc_gather_body(emb, bidx, outs, idxv, buf, sem):
    cid = lax.axis_index("c")
    sid = lax.axis_index("s")
    wid = sid * NC + cid
    o = wid * BT
    for b in range(3):
        pltpu.sync_copy(bidx.at[b, pl.ds(o, BT)], idxv)
        pltpu.async_copy(emb.at[idxv], buf, sem).wait()
        pltpu.sync_copy(buf, outs.at[b, pl.ds(o, BT)])


_sc_gather = functools.partial(
    pl.kernel,
    out_type=jax.ShapeDtypeStruct((3, B, DO), jnp.float32),
    mesh=_mesh,
    scratch_types=[
        pltpu.VMEM((BT,), jnp.int32),
        pltpu.VMEM((BT, DO), jnp.float32),
        pltpu.SemaphoreType.DMA,
    ],
)(_sc_gather_body)


def _tc_loss_body(o_ref, wd_ref, out_ref):
    o1 = o_ref[0]
    o2 = o_ref[1]
    o3 = o_ref[2]
    y_ui = jnp.sum(o1 * o2, axis=1)
    y_uj = jnp.sum(o1 * o3, axis=1)
    d = y_ui - y_uj
    sig = 1.0 / (1.0 + jnp.exp(-d))
    loss = jnp.sum(-jnp.log(sig + 1e-12))
    loss = loss + WD * 0.5 * wd_ref[0, 0]
    out_ref[...] = (loss / B).reshape(1, 1)


_tc_loss = pl.pallas_call(
    _tc_loss_body,
    out_shape=jax.ShapeDtypeStruct((1, 1), jnp.float32),
)


def kernel(feats, W, edge_index, batch1, batch2, batch3):
    src = edge_index[0]
    dst = edge_index[1]
    pad = NW * EWP - E
    srcp = jnp.concatenate([src, jnp.zeros((pad,), jnp.int32)]).reshape(NW, KW, CH)
    dstp = jnp.concatenate([dst, jnp.full((pad,), DUMMY, jnp.int32)]).reshape(NW, KW, CH)
    zer = jnp.zeros((RT,), jnp.float32)
    z128 = jnp.zeros((CH, D), jnp.float32)
    ones = jnp.ones((CH,), jnp.float32)
    feats2, aggp, rs = _sc_msg(feats, srcp, dstp, zer, z128, ones)
    del feats2
    Wp = jnp.zeros((D, DO), jnp.float32).at[:, :DREAL].set(W)
    emb, wd = _tc_embed(aggp, rs, Wp)
    bidx = jnp.stack([batch1, batch2, batch3])
    outs = _sc_gather(emb, bidx)
    loss = _tc_loss(outs, wd)
    return loss[0, 0]


# speedup vs baseline: 1.4176x; 1.0202x over previous
import functools

import jax
import jax.numpy as jnp
from jax import lax
from jax.experimental import pallas as pl
from jax.experimental.pallas import tpu as pltpu
import jax.experimental.pallas.tpu_sc as plsc

N = 10000
E = 320000
D = 128
DO = 128
DREAL = 50
B = 1024
WD = 5e-4

NC, NS = 2, 16
NW = NC * NS
RT = 640
NPAD = RT * NS
DUMMY = N + 8
CH = 128
KW = 79
EWP = KW * CH
BT = B // NW

_mesh = plsc.VectorSubcoreMesh(core_axis_name="c", subcore_axis_name="s",
                               num_cores=NC, num_subcores=NS)


def _sc_msg_body(feats, srcp, dstp, zer, z128, ones,
                 feats2, aggp, rs_out,
                 src_v, dst_v, rows_v, fbuf, degv, rsv, ones_v, zer_v,
                 agg_s, deg_s, sem):
    cid = lax.axis_index("c")
    sid = lax.axis_index("s")
    wid = sid * NC + cid
    r0 = sid * RT

    pltpu.sync_copy(zer, zer_v)
    pltpu.sync_copy(ones, ones_v)
    pltpu.sync_copy(z128, rows_v)
    pltpu.sync_copy(zer_v, deg_s.at[pl.ds(r0, RT)])
    for j in range(RT // CH):
        pltpu.sync_copy(rows_v, agg_s.at[pl.ds(r0 + j * CH, CH)])
    plsc.subcore_barrier()

    for w0 in (0, NS):
        pltpu.sync_copy(dstp.at[sid + w0], dst_v)

        @pl.loop(0, KW)
        def _deg(k):
            pltpu.async_copy(ones_v, deg_s.at[dst_v.at[k]], sem, add=True)

        @pl.loop(0, KW)
        def _drain(k):
            pltpu.make_async_copy(ones_v, deg_s.at[dst_v.at[0]], sem).wait()
    plsc.subcore_barrier()

    pltpu.sync_copy(deg_s.at[pl.ds(r0, RT)], degv)

    @pl.loop(0, RT // 16)
    def _rs(i):
        x = jnp.maximum(degv[pl.ds(i * 16, 16)], 1.0)
        h = jnp.int32(0x5F3759DF) - (lax.bitcast_convert_type(x, jnp.int32) >> 1)
        y = lax.bitcast_convert_type(h, jnp.float32)
        y = y * (1.5 - 0.5 * x * y * y)
        y = y * (1.5 - 0.5 * x * y * y)
        y = y * (1.5 - 0.5 * x * y * y)
        rsv[pl.ds(i * 16, 16)] = y

    pltpu.sync_copy(rsv, rs_out.at[cid, pl.ds(r0, RT)])

    nrows = jnp.minimum(RT, N - r0)

    @pl.loop(0, nrows // 16)
    def _scale(k):
        g0 = r0 + k * 16
        pltpu.sync_copy(feats.at[pl.ds(g0, 16)], fbuf)
        rv = rsv[pl.ds(k * 16, 16)]
        for r in range(16):
            sv = lax.broadcast(rv[r], (16,))
            for v in range(D // 16):
                fbuf[r, pl.ds(v * 16, 16)] = fbuf[r, pl.ds(v * 16, 16)] * sv
        pltpu.sync_copy(fbuf, feats2.at[pl.ds(g0, 16)])
    plsc.subcore_barrier()

    pltpu.sync_copy(srcp.at[wid], src_v)
    pltpu.sync_copy(dstp.at[wid], dst_v)

    @pl.loop(0, KW)
    def _edge(k):
        pltpu.async_copy(feats2.at[src_v.at[k]], rows_v, sem).wait()
        pltpu.sync_copy(rows_v, agg_s.at[dst_v.at[k]], add=True)
    plsc.subcore_barrier()

    for j in range(RT // CH):
        pltpu.sync_copy(agg_s.at[pl.ds(r0 + j * CH, CH)], rows_v)
        pltpu.sync_copy(rows_v, aggp.at[cid, pl.ds(r0 + j * CH, CH)])


_sc_msg = functools.partial(
    pl.kernel,
    out_type=(
        jax.ShapeDtypeStruct((N, D), jnp.float32),
        jax.ShapeDtypeStruct((NC, NPAD, D), jnp.float32),
        jax.ShapeDtypeStruct((NC, NPAD), jnp.float32),
    ),
    mesh=_mesh,
    scratch_types=[
        pltpu.VMEM((KW, CH), jnp.int32),
        pltpu.VMEM((KW, CH), jnp.int32),
        pltpu.VMEM((CH, D), jnp.float32),
        pltpu.VMEM((16, D), jnp.float32),
        pltpu.VMEM((RT,), jnp.float32),
        pltpu.VMEM((RT,), jnp.float32),
        pltpu.VMEM((CH,), jnp.float32),
        pltpu.VMEM((RT,), jnp.float32),
        pltpu.VMEM_SHARED((NPAD, D), jnp.float32),
        pltpu.VMEM_SHARED((NPAD,), jnp.float32),
        pltpu.SemaphoreType.DMA,
    ],
)(_sc_msg_body)


def _tc_embed_body(aggp_ref, rs_ref, w_ref, emb_ref, wd_ref):
    a = aggp_ref[0, :N, :] + aggp_ref[1, :N, :]
    a = a * rs_ref[0, :N][:, None]
    h = jnp.tanh(jnp.dot(a, w_ref[...], preferred_element_type=jnp.float32))
    ss = jnp.sum(h * h, axis=1, keepdims=True)
    e = h * lax.rsqrt(ss + 1e-12)
    emb_ref[...] = e
    wd_ref[...] = jnp.sum(e * e).reshape(1, 1)


_tc_embed = pl.pallas_call(
    _tc_embed_body,
    out_shape=(
        jax.ShapeDtypeStruct((N, DO), jnp.float32),
        jax.ShapeDtypeStruct((1, 1), jnp.float32),
    ),
)


def _sc_gather_body(emb, bidx, outs, idxv, buf, sem):
    cid = lax.axis_index("c")
    sid = lax.axis_index("s")
    wid = sid * NC + cid
    o = wid * BT
    for b in range(3):
        pltpu.sync_copy(bidx.at[b, pl.ds(o, BT)], idxv)
        pltpu.async_copy(emb.at[idxv], buf, sem).wait()
        pltpu.sync_copy(buf, outs.at[b, pl.ds(o, BT)])


_sc_gather = functools.partial(
    pl.kernel,
    out_type=jax.ShapeDtypeStruct((3, B, DO), jnp.float32),
    mesh=_mesh,
    scratch_types=[
        pltpu.VMEM((BT,), jnp.int32),
        pltpu.VMEM((BT, DO), jnp.float32),
        pltpu.SemaphoreType.DMA,
    ],
)(_sc_gather_body)


def _tc_loss_body(o_ref, wd_ref, out_ref):
    o1 = o_ref[0]
    o2 = o_ref[1]
    o3 = o_ref[2]
    y_ui = jnp.sum(o1 * o2, axis=1)
    y_uj = jnp.sum(o1 * o3, axis=1)
    d = y_ui - y_uj
    sig = 1.0 / (1.0 + jnp.exp(-d))
    loss = jnp.sum(-jnp.log(sig + 1e-12))
    loss = loss + WD * 0.5 * wd_ref[0, 0]
    out_ref[...] = (loss / B).reshape(1, 1)


_tc_loss = pl.pallas_call(
    _tc_loss_body,
    out_shape=jax.ShapeDtypeStruct((1, 1), jnp.float32),
)


def kernel(feats, W, edge_index, batch1, batch2, batch3):
    src = edge_index[0]
    dst = edge_index[1]
    pad = NW * EWP - E
    srcp = jnp.concatenate([src, jnp.zeros((pad,), jnp.int32)]).reshape(NW, KW, CH)
    dstp = jnp.concatenate([dst, jnp.full((pad,), DUMMY, jnp.int32)]).reshape(NW, KW, CH)
    zer = jnp.zeros((RT,), jnp.float32)
    z128 = jnp.zeros((CH, D), jnp.float32)
    ones = jnp.ones((CH,), jnp.float32)
    feats2, aggp, rs = _sc_msg(feats, srcp, dstp, zer, z128, ones)
    del feats2
    Wp = jnp.zeros((D, DO), jnp.float32).at[:, :DREAL].set(W)
    emb, wd = _tc_embed(aggp, rs, Wp)
    bidx = jnp.stack([batch1, batch2, batch3])
    outs = _sc_gather(emb, bidx)
    loss = _tc_loss(outs, wd)
    return loss[0, 0]
